# baseline (speedup 1.0000x reference)
import jax
import jax.numpy as jnp
from jax import lax
from jax.experimental import pallas as pl
from jax.experimental.pallas import tpu as pltpu
from jax.experimental.pallas import tpu_sc as plsc

ENC = 4096
L = 16
SUB = ENC // L
ROWS = 100
CHUNK = 10


def _row_gather_body(mat_hbm, pa_hbm, out_hbm, pa_v, buf_v, sem):
    c = lax.axis_index("c")
    s = lax.axis_index("s")

    @pl.when(jnp.logical_and(c == 0, s < ROWS // CHUNK))
    def _():
        start = s * CHUNK
        chunk = pltpu.async_copy(
            mat_hbm.at[pl.ds(start * L, CHUNK * L)], buf_v, sem
        )
        pltpu.sync_copy(pa_hbm, pa_v)
        pa_vec = pa_v[0, :] * 10 + pa_v[1, :]
        row = pa_vec[0]
        chunk.wait()

        @pl.when(jnp.logical_and(row >= start, row < start + CHUNK))
        def _():
            pltpu.sync_copy(buf_v.at[pl.ds((row - start) * L, L)], out_hbm)


_row_gather = pl.kernel(
    _row_gather_body,
    mesh=plsc.VectorSubcoreMesh(
        core_axis_name="c", subcore_axis_name="s", num_cores=1
    ),
    out_type=jax.ShapeDtypeStruct((L, SUB), jnp.float32),
    scratch_types=[
        pltpu.VMEM((2, L), jnp.int32),
        pltpu.VMEM((CHUNK * L, SUB), jnp.float32),
        pltpu.SemaphoreType.DMA,
    ],
)


def kernel(x, parameters_encoding_matrix, p, a):
    del x
    mat = parameters_encoding_matrix.reshape(ROWS * L, SUB)
    pi = jnp.full((1, L), p, dtype=jnp.int32)
    ai = jnp.full((1, L), a, dtype=jnp.int32)
    pa = jnp.concatenate([pi, ai], axis=0)
    out = _row_gather(mat, pa)
    return out.reshape(1, ENC, 1)


# speedup vs baseline: 1.0480x; 1.0480x over previous
import jax
import jax.numpy as jnp
from jax import lax
from jax.experimental import pallas as pl
from jax.experimental.pallas import tpu as pltpu
from jax.experimental.pallas import tpu_sc as plsc

ENC = 4096
L = 16
SUB = ENC // L


def _row_gather_body(mat_hbm, pa_hbm, out_hbm, pa_v, rows_v):
    c = lax.axis_index("c")
    s = lax.axis_index("s")

    @pl.when(jnp.logical_and(c == 0, s == 0))
    def _():
        pltpu.sync_copy(pa_hbm, pa_v)
        pa_vec = pa_v[0, :] * 10 + pa_v[1, :]
        row = pa_vec[0]
        pltpu.sync_copy(mat_hbm.at[pl.ds(row * (ENC // SUB), L)], rows_v)
        pltpu.sync_copy(rows_v, out_hbm)


_row_gather = pl.kernel(
    _row_gather_body,
    mesh=plsc.VectorSubcoreMesh(
        core_axis_name="c", subcore_axis_name="s", num_cores=1
    ),
    out_type=jax.ShapeDtypeStruct((L, SUB), jnp.float32),
    scratch_types=[
        pltpu.VMEM((2, L), jnp.int32),
        pltpu.VMEM((L, SUB), jnp.float32),
    ],
    compiler_params=pltpu.CompilerParams(skip_device_barrier=True),
)


def kernel(x, parameters_encoding_matrix, p, a):
    del x
    mat = parameters_encoding_matrix.reshape(-1, SUB)
    pi = jnp.full((1, L), p, dtype=jnp.int32)
    ai = jnp.full((1, L), a, dtype=jnp.int32)
    pa = jnp.concatenate([pi, ai], axis=0)
    out = _row_gather(mat, pa)
    return out.reshape(1, ENC, 1)
